# initial kernel scaffold (unmeasured)
import jax
import jax.numpy as jnp
from jax import lax
from jax.experimental import pallas as pl
from jax.experimental.pallas import tpu as pltpu


def kernel(
    x,
):
    def body(*refs):
        pass

    out_shape = jax.ShapeDtypeStruct(..., jnp.float32)
    return pl.pallas_call(body, out_shape=out_shape)(...)


# baseline (device time: 806183 ns/iter reference)
import jax
import jax.numpy as jnp
from jax import lax
from jax.experimental import pallas as pl
from jax.experimental.pallas import tpu as pltpu

ROWS = 4096
COLS = 2048
N_CHUNK = 8
CHUNK = ROWS // N_CHUNK


def kernel(x):
    def body(x_hbm, out_hbm, recv_vmem, x_chunk, sum_chunk,
             local_sems, send_sems, recv_sems):
        my_x = lax.axis_index("x")
        my_y = lax.axis_index("y")
        col = my_y * COLS

        barrier = pltpu.get_barrier_semaphore()
        pl.semaphore_signal(barrier, inc=1, device_id=(1 - my_x, my_y),
                            device_id_type=pl.DeviceIdType.MESH)
        pl.semaphore_signal(barrier, inc=1, device_id=(my_x, 1 - my_y),
                            device_id_type=pl.DeviceIdType.MESH)
        pl.semaphore_wait(barrier, 2)

        rdma1 = pltpu.make_async_remote_copy(
            src_ref=x_hbm.at[0],
            dst_ref=recv_vmem,
            send_sem=send_sems.at[0],
            recv_sem=recv_sems.at[0],
            device_id=(1 - my_x, my_y),
            device_id_type=pl.DeviceIdType.MESH,
        )
        rdma1.start()
        rdma1.wait()

        for c in range(N_CHUNK):
            r = pl.ds(c * CHUNK, CHUNK)
            cp_in = pltpu.make_async_copy(x_hbm.at[0, r, :], x_chunk,
                                          local_sems.at[0])
            cp_in.start()
            cp_in.wait()
            sum_chunk[...] = x_chunk[...] + recv_vmem[r, :]
            cp_out = pltpu.make_async_copy(
                sum_chunk, out_hbm.at[r, pl.ds(col, COLS)], local_sems.at[1])
            cp_out.start()
            cp_out.wait()

        rdma2 = pltpu.make_async_remote_copy(
            src_ref=out_hbm.at[:, pl.ds(col, COLS)],
            dst_ref=out_hbm.at[:, pl.ds(col, COLS)],
            send_sem=send_sems.at[1],
            recv_sem=recv_sems.at[1],
            device_id=(my_x, 1 - my_y),
            device_id_type=pl.DeviceIdType.MESH,
        )
        rdma2.start()
        rdma2.wait()

    return pl.pallas_call(
        body,
        out_shape=jax.ShapeDtypeStruct((ROWS, 2 * COLS), jnp.float32),
        in_specs=[pl.BlockSpec(memory_space=pl.ANY)],
        out_specs=pl.BlockSpec(memory_space=pl.ANY),
        scratch_shapes=[
            pltpu.VMEM((ROWS, COLS), jnp.float32),
            pltpu.VMEM((CHUNK, COLS), jnp.float32),
            pltpu.VMEM((CHUNK, COLS), jnp.float32),
            pltpu.SemaphoreType.DMA((2,)),
            pltpu.SemaphoreType.DMA((2,)),
            pltpu.SemaphoreType.DMA((2,)),
        ],
        compiler_params=pltpu.CompilerParams(
            collective_id=0, vmem_limit_bytes=56 * 1024 * 1024
        ),
    )(x)


# device time: 433495 ns/iter; 1.8597x vs baseline; 1.8597x over previous
import jax
import jax.numpy as jnp
from jax import lax
from jax.experimental import pallas as pl
from jax.experimental.pallas import tpu as pltpu

ROWS = 4096
COLS = 2048
N_CHUNK = 16
CHUNK = ROWS // N_CHUNK


def kernel(x):
    def body(x_hbm, out_hbm, recv_vmem, x_buf, sum_buf,
             load_sems, store_sems, send1, recv1, send2, recv2):
        my_x = lax.axis_index("x")
        my_y = lax.axis_index("y")
        col = pl.ds(my_y * COLS, COLS)

        barrier = pltpu.get_barrier_semaphore()
        pl.semaphore_signal(barrier, inc=1, device_id=(1 - my_x, my_y),
                            device_id_type=pl.DeviceIdType.MESH)
        pl.semaphore_signal(barrier, inc=1, device_id=(my_x, 1 - my_y),
                            device_id_type=pl.DeviceIdType.MESH)
        pl.semaphore_wait(barrier, 2)

        rdma1 = []
        for c in range(N_CHUNK):
            r = pl.ds(c * CHUNK, CHUNK)
            d = pltpu.make_async_remote_copy(
                src_ref=x_hbm.at[0, r, :],
                dst_ref=recv_vmem.at[r, :],
                send_sem=send1.at[c],
                recv_sem=recv1.at[c],
                device_id=(1 - my_x, my_y),
                device_id_type=pl.DeviceIdType.MESH,
            )
            d.start()
            rdma1.append(d)

        loads = [None, None]
        ld = pltpu.make_async_copy(
            x_hbm.at[0, pl.ds(0, CHUNK), :], x_buf.at[0], load_sems.at[0])
        ld.start()
        loads[0] = ld

        stores = [None, None]
        rdma2 = []
        for c in range(N_CHUNK):
            s = c % 2
            r = pl.ds(c * CHUNK, CHUNK)
            if c + 1 < N_CHUNK:
                sn = (c + 1) % 2
                ldn = pltpu.make_async_copy(
                    x_hbm.at[0, pl.ds((c + 1) * CHUNK, CHUNK), :],
                    x_buf.at[sn], load_sems.at[sn])
                ldn.start()
                loads[sn] = ldn
            loads[s].wait()
            if stores[s] is not None:
                stores[s][0].wait()
                stores[s][1].wait_send()
            rdma1[c].wait_recv()
            sum_buf[s] = x_buf[s] + recv_vmem[r, :]
            st = pltpu.make_async_copy(
                sum_buf.at[s], out_hbm.at[r, col], store_sems.at[s])
            st.start()
            d2 = pltpu.make_async_remote_copy(
                src_ref=sum_buf.at[s],
                dst_ref=out_hbm.at[r, col],
                send_sem=send2.at[c],
                recv_sem=recv2.at[c],
                device_id=(my_x, 1 - my_y),
                device_id_type=pl.DeviceIdType.MESH,
            )
            d2.start()
            stores[s] = (st, d2)
            rdma2.append(d2)

        for s in range(2):
            stores[s][0].wait()
            stores[s][1].wait_send()
        for c in range(N_CHUNK):
            rdma1[c].wait_send()
        for c in range(N_CHUNK):
            rdma2[c].wait_recv()

    return pl.pallas_call(
        body,
        out_shape=jax.ShapeDtypeStruct((ROWS, 2 * COLS), jnp.float32),
        in_specs=[pl.BlockSpec(memory_space=pl.ANY)],
        out_specs=pl.BlockSpec(memory_space=pl.ANY),
        scratch_shapes=[
            pltpu.VMEM((ROWS, COLS), jnp.float32),
            pltpu.VMEM((2, CHUNK, COLS), jnp.float32),
            pltpu.VMEM((2, CHUNK, COLS), jnp.float32),
            pltpu.SemaphoreType.DMA((2,)),
            pltpu.SemaphoreType.DMA((2,)),
            pltpu.SemaphoreType.DMA((N_CHUNK,)),
            pltpu.SemaphoreType.DMA((N_CHUNK,)),
            pltpu.SemaphoreType.DMA((N_CHUNK,)),
            pltpu.SemaphoreType.DMA((N_CHUNK,)),
        ],
        compiler_params=pltpu.CompilerParams(
            collective_id=0, vmem_limit_bytes=56 * 1024 * 1024
        ),
    )(x)
